# baseline (speedup 1.0000x reference)
import functools

import jax
import jax.numpy as jnp
import numpy as np
from jax import lax
from jax.experimental import pallas as pl
from jax.experimental.pallas import tpu as pltpu
from jax.experimental.pallas import tpu_sc as plsc

N = 10000
NPAD = 10240
E = 320000
D_IN = 128
H1 = 256
H2 = 64
C_OUT = 40

NC = 2
NS = 16
NW = NC * NS
K = 64
E2 = 327680
EPT = E2 // NW
NCH = EPT // K
PADN = E2 - E
REAL_LAST = EPT - PADN
RPT = NPAD // NS

RB = 1024

_PAD_NP = np.asarray(N + np.arange(PADN) % (NPAD - N), dtype=np.int32)


def _stage_indices(idx_hbm, pad_hbm, buf, w):
    @pl.when(w < NW - 1)
    def _all_real():
        pltpu.sync_copy(idx_hbm.at[pl.ds(w * EPT, EPT)], buf)

    @pl.when(w == NW - 1)
    def _with_pad():
        pltpu.sync_copy(idx_hbm.at[pl.ds(w * EPT, REAL_LAST)],
                        buf.at[pl.ds(0, REAL_LAST)])
        pltpu.sync_copy(pad_hbm, buf.at[pl.ds(REAL_LAST, PADN)])


DEG_W = 16
DEG_LAG = 8


def _deg_body(dst_hbm, pad_hbm, out_hbm, acc, dbuf, ones_rows, zbuf, asem):
    c = lax.axis_index("c")
    s = lax.axis_index("s")
    w = c * NS + s
    r0 = s * RPT

    ones16 = jnp.ones((16,), jnp.float32)
    zeros16 = jnp.zeros((16,), jnp.float32)

    def fill_step(j, carry):
        ones_rows[j] = ones16
        return carry

    lax.fori_loop(0, K, fill_step, 0)

    def zfill_step(j, carry):
        zbuf[j] = zeros16
        return carry

    lax.fori_loop(0, 64, zfill_step, 0)

    def zcopy_step(j, carry):
        pltpu.sync_copy(zbuf, acc.at[pl.ds(r0 + j * 64, 64)])
        return carry

    lax.fori_loop(0, RPT // 64, zcopy_step, 0)

    _stage_indices(dst_hbm, pad_hbm, dbuf, w)
    plsc.subcore_barrier()

    def add_step(i, carry):
        pltpu.async_copy(ones_rows, acc.at[dbuf.at[pl.ds(i * K, K)]], asem,
                         add=True)

        @pl.when(i >= DEG_LAG)
        def _drain_one():
            pltpu.make_async_copy(ones_rows, acc.at[dbuf.at[pl.ds(0, K)]],
                                  asem).wait()

        return carry

    lax.fori_loop(0, NCH, add_step, 0)

    def drain_step(i, carry):
        pltpu.make_async_copy(ones_rows, acc.at[dbuf.at[pl.ds(0, K)]],
                              asem).wait()
        return carry

    lax.fori_loop(0, DEG_LAG, drain_step, 0)

    plsc.subcore_barrier()
    pltpu.sync_copy(acc.at[pl.ds(r0, RPT)], out_hbm.at[c, pl.ds(r0, RPT)])


_deg_kernel = functools.partial(
    pl.kernel,
    out_type=jax.ShapeDtypeStruct((NC, NPAD, DEG_W), jnp.float32),
    mesh=plsc.VectorSubcoreMesh(core_axis_name="c", subcore_axis_name="s"),
    compiler_params=pltpu.CompilerParams(use_tc_tiling_on_sc=False),
    scratch_types=[
        pltpu.VMEM_SHARED((NPAD, DEG_W), jnp.float32),
        pltpu.VMEM((EPT,), jnp.int32),
        pltpu.VMEM((K, DEG_W), jnp.float32),
        pltpu.VMEM((64, DEG_W), jnp.float32),
        pltpu.SemaphoreType.DMA,
    ],
)(_deg_body)


def _make_propagate(D):
    def body(y_hbm, src_hbm, dst_hbm, pad_hbm, out_hbm,
             acc, sidx, didx, rows0, rows1, gsem0, gsem1):
        c = lax.axis_index("c")
        s = lax.axis_index("s")
        w = c * NS + s
        r0 = s * RPT

        zeros16 = jnp.zeros((16,), jnp.float32)

        def zfill_step(q, carry):
            rows0[q // (D // 16), pl.ds((q % (D // 16)) * 16, 16)] = zeros16
            return carry

        lax.fori_loop(0, K * D // 16, zfill_step, 0)

        def zcopy_step(q, carry):
            pltpu.sync_copy(rows0, acc.at[pl.ds(r0 + q * K, K)])
            return carry

        lax.fori_loop(0, RPT // K, zcopy_step, 0)

        _stage_indices(src_hbm, pad_hbm, sidx, w)
        _stage_indices(dst_hbm, pad_hbm, didx, w)
        plsc.subcore_barrier()

        rows = (rows0, rows1)
        gsem = (gsem0, gsem1)
        pltpu.async_copy(y_hbm.at[sidx.at[pl.ds(0, K)]], rows0, gsem0)
        pltpu.async_copy(y_hbm.at[sidx.at[pl.ds(K, K)]], rows1, gsem1)

        def pair_step(p, carry):
            for b in range(2):
                i = 2 * p + b
                pltpu.make_async_copy(
                    y_hbm.at[sidx.at[pl.ds(i * K, K)]], rows[b],
                    gsem[b]).wait()
                pltpu.sync_copy(rows[b], acc.at[didx.at[pl.ds(i * K, K)]],
                                add=True)

                @pl.when(i + 2 < NCH)
                def _next_gather():
                    pltpu.async_copy(
                        y_hbm.at[sidx.at[pl.ds((i + 2) * K, K)]], rows[b],
                        gsem[b])

            return carry

        lax.fori_loop(0, NCH // 2, pair_step, 0)

        plsc.subcore_barrier()
        pltpu.sync_copy(acc.at[pl.ds(r0, RPT)], out_hbm.at[c, pl.ds(r0, RPT)])

    return functools.partial(
        pl.kernel,
        out_type=jax.ShapeDtypeStruct((NC, NPAD, D), jnp.float32),
        mesh=plsc.VectorSubcoreMesh(core_axis_name="c", subcore_axis_name="s"),
        compiler_params=pltpu.CompilerParams(use_tc_tiling_on_sc=False),
        scratch_types=[
            pltpu.VMEM_SHARED((NPAD, D), jnp.float32),
            pltpu.VMEM((EPT,), jnp.int32),
            pltpu.VMEM((EPT,), jnp.int32),
            pltpu.VMEM((K, D), jnp.float32),
            pltpu.VMEM((K, D), jnp.float32),
            pltpu.SemaphoreType.DMA,
            pltpu.SemaphoreType.DMA,
        ],
    )(body)


_prop128 = _make_propagate(D_IN)
_prop64 = _make_propagate(H2)


def _prescale_body(cnt_ref, x_ref, dinv_ref, y_ref):
    deg = 1.0 + cnt_ref[0, :, 0:1] + cnt_ref[1, :, 0:1]
    dv = lax.rsqrt(deg)
    dinv_ref[...] = dv
    y_ref[...] = x_ref[...] * dv


_prescale = pl.pallas_call(
    _prescale_body,
    grid=(NPAD // RB,),
    in_specs=[
        pl.BlockSpec((NC, RB, DEG_W), lambda i: (0, i, 0)),
        pl.BlockSpec((RB, D_IN), lambda i: (i, 0)),
    ],
    out_specs=[
        pl.BlockSpec((RB, 1), lambda i: (i, 0)),
        pl.BlockSpec((RB, D_IN), lambda i: (i, 0)),
    ],
    out_shape=[
        jax.ShapeDtypeStruct((NPAD, 1), jnp.float32),
        jax.ShapeDtypeStruct((NPAD, D_IN), jnp.float32),
    ],
)


def _dense1_body(a_ref, y_ref, dinv_ref, w1_ref, b1_ref, w2_ref, out_ref):
    dv = dinv_ref[...]
    p = dv * (a_ref[0] + a_ref[1] + y_ref[...])
    z = jnp.dot(p, w1_ref[...], preferred_element_type=jnp.float32) + b1_ref[...]
    h = jnp.maximum(z, 0.0)
    g = jnp.dot(h, w2_ref[...], preferred_element_type=jnp.float32)
    out_ref[...] = dv * g


_dense1 = pl.pallas_call(
    _dense1_body,
    grid=(NPAD // RB,),
    in_specs=[
        pl.BlockSpec((NC, RB, D_IN), lambda i: (0, i, 0)),
        pl.BlockSpec((RB, D_IN), lambda i: (i, 0)),
        pl.BlockSpec((RB, 1), lambda i: (i, 0)),
        pl.BlockSpec((D_IN, H1), lambda i: (0, 0)),
        pl.BlockSpec((1, H1), lambda i: (0, 0)),
        pl.BlockSpec((H1, H2), lambda i: (0, 0)),
    ],
    out_specs=pl.BlockSpec((RB, H2), lambda i: (i, 0)),
    out_shape=jax.ShapeDtypeStruct((NPAD, H2), jnp.float32),
)


def _dense2_body(a_ref, y_ref, dinv_ref, b2_ref, wfc_ref, bfc_ref, out_ref):
    dv = dinv_ref[...]
    z = dv * (a_ref[0] + a_ref[1] + y_ref[...]) + b2_ref[...]
    out_ref[...] = (
        jnp.dot(z, wfc_ref[...], preferred_element_type=jnp.float32) + bfc_ref[...]
    )


RB2 = 1000

_dense2 = pl.pallas_call(
    _dense2_body,
    grid=(N // RB2,),
    in_specs=[
        pl.BlockSpec((NC, RB2, H2), lambda i: (0, i, 0)),
        pl.BlockSpec((RB2, H2), lambda i: (i, 0)),
        pl.BlockSpec((RB2, 1), lambda i: (i, 0)),
        pl.BlockSpec((1, H2), lambda i: (0, 0)),
        pl.BlockSpec((H2, C_OUT), lambda i: (0, 0)),
        pl.BlockSpec((1, C_OUT), lambda i: (0, 0)),
    ],
    out_specs=pl.BlockSpec((RB2, C_OUT), lambda i: (i, 0)),
    out_shape=jax.ShapeDtypeStruct((N, C_OUT), jnp.float32),
)


def kernel(x, edge_index, W1, b1, W2, b2, Wfc, bfc):
    ei = edge_index.astype(jnp.int32)
    src = ei[0]
    dst = ei[1]
    padc = jnp.asarray(_PAD_NP)
    xp = jnp.pad(x, ((0, NPAD - N), (0, 0)))

    counts = _deg_kernel(dst, padc)
    dinv, y1 = _prescale(counts, xp)

    prop1 = _prop128(y1, src, dst, padc)
    y2 = _dense1(prop1, y1, dinv, W1, b1.reshape(1, H1), W2)

    prop2 = _prop64(y2, src, dst, padc)
    out = _dense2(prop2, y2, dinv, b2.reshape(1, H2), Wfc, bfc.reshape(1, C_OUT))
    return out


# speedup vs baseline: 1.2094x; 1.2094x over previous
import functools

import jax
import jax.numpy as jnp
from jax import lax
from jax.experimental import pallas as pl
from jax.experimental.pallas import tpu as pltpu
from jax.experimental.pallas import tpu_sc as plsc

N = 10000
NPAD = 10240
E = 320000
D_IN = 128
H1 = 256
H2 = 64
C_OUT = 40

NC = 2
NS = 16
NW = NC * NS
K = 128
E2 = 327680
EPT = E2 // NW
NCH = EPT // K
CPB = 8
NBLK = NCH // CPB
EROWS = E2 // K
RPT = NPAD // NS

RB = 2048


DEG_W = 16


def _deg_body(dst_hbm, out_hbm, acc, dblk0, dblk1, ones_rows, zbuf,
              isem0, isem1, asem):
    c = lax.axis_index("c")
    s = lax.axis_index("s")
    wid = c * NS + s
    r0 = s * RPT
    row0 = wid * NCH

    ones16 = jnp.ones((16,), jnp.float32)
    zeros16 = jnp.zeros((16,), jnp.float32)

    def fill_step(j, carry):
        ones_rows[j] = ones16
        return carry

    lax.fori_loop(0, K, fill_step, 0)

    def zfill_step(j, carry):
        zbuf[j] = zeros16
        return carry

    lax.fori_loop(0, 64, zfill_step, 0)

    def zcopy_step(j, carry):
        pltpu.sync_copy(zbuf, acc.at[pl.ds(r0 + j * 64, 64)])
        return carry

    lax.fori_loop(0, RPT // 64, zcopy_step, 0)

    pltpu.sync_copy(dst_hbm.at[pl.ds(row0, CPB)], dblk0)
    pltpu.async_copy(dst_hbm.at[pl.ds(row0 + CPB, CPB)], dblk1, isem1)
    plsc.subcore_barrier()

    dblk = (dblk0, dblk1)
    isem = (isem0, isem1)

    def pair_step(p, carry):
        for half in range(2):
            j = 2 * p + half
            blk = dblk[half]
            sem = isem[half]

            @pl.when(j > 0)
            def _wait_blk():
                pltpu.make_async_copy(
                    dst_hbm.at[pl.ds(row0 + j * CPB, CPB)], blk, sem).wait()

            for r in range(CPB):
                pltpu.async_copy(ones_rows, acc.at[blk.at[r]], asem, add=True)
            for r in range(CPB):
                pltpu.make_async_copy(ones_rows, acc.at[blk.at[r]], asem).wait()

            @pl.when(j + 2 < NBLK)
            def _load_next():
                pltpu.async_copy(
                    dst_hbm.at[pl.ds(row0 + (j + 2) * CPB, CPB)], blk, sem)

        return carry

    lax.fori_loop(0, NBLK // 2, pair_step, 0)

    plsc.subcore_barrier()
    pltpu.sync_copy(acc.at[pl.ds(r0, RPT)], out_hbm.at[c, pl.ds(r0, RPT)])


_deg_kernel = functools.partial(
    pl.kernel,
    out_type=jax.ShapeDtypeStruct((NC, NPAD, DEG_W), jnp.float32),
    mesh=plsc.VectorSubcoreMesh(core_axis_name="c", subcore_axis_name="s"),
    compiler_params=pltpu.CompilerParams(use_tc_tiling_on_sc=False),
    scratch_types=[
        pltpu.VMEM_SHARED((NPAD, DEG_W), jnp.float32),
        pltpu.VMEM((CPB, K), jnp.int32),
        pltpu.VMEM((CPB, K), jnp.int32),
        pltpu.VMEM((K, DEG_W), jnp.float32),
        pltpu.VMEM((64, DEG_W), jnp.float32),
        pltpu.SemaphoreType.DMA,
        pltpu.SemaphoreType.DMA,
        pltpu.SemaphoreType.DMA,
    ],
)(_deg_body)


def _make_propagate(D):
    def body(y_hbm, src_hbm, dst_hbm, out_hbm,
             acc, sblk0, sblk1, dblk0, dblk1, rows0, rows1,
             isem0, isem1, gsem0, gsem1):
        c = lax.axis_index("c")
        s = lax.axis_index("s")
        wid = c * NS + s
        r0 = s * RPT
        row0 = wid * NCH

        zeros16 = jnp.zeros((16,), jnp.float32)

        def zfill_step(q, carry):
            rows0[q // (D // 16), pl.ds((q % (D // 16)) * 16, 16)] = zeros16
            return carry

        lax.fori_loop(0, K * D // 16, zfill_step, 0)

        def zcopy_step(q, carry):
            pltpu.sync_copy(rows0, acc.at[pl.ds(r0 + q * K, K)])
            return carry

        lax.fori_loop(0, RPT // K, zcopy_step, 0)

        sblk = (sblk0, sblk1)
        dblk = (dblk0, dblk1)
        isem = (isem0, isem1)
        rows = (rows0, rows1)
        gsem = (gsem0, gsem1)

        pltpu.sync_copy(src_hbm.at[pl.ds(row0, CPB)], sblk0)
        pltpu.sync_copy(dst_hbm.at[pl.ds(row0, CPB)], dblk0)
        pltpu.async_copy(src_hbm.at[pl.ds(row0 + CPB, CPB)], sblk1, isem1)
        pltpu.async_copy(dst_hbm.at[pl.ds(row0 + CPB, CPB)], dblk1, isem1)
        plsc.subcore_barrier()
        pltpu.async_copy(y_hbm.at[sblk0.at[0]], rows0, gsem0)
        pltpu.async_copy(y_hbm.at[sblk0.at[1]], rows1, gsem1)

        def pair_step(p, carry):
            for half in range(2):
                j = 2 * p + half
                cb = sblk[half]
                db = dblk[half]
                nb = sblk[1 - half]
                ndb = dblk[1 - half]
                for r in range(CPB):
                    b = r % 2
                    pltpu.make_async_copy(
                        y_hbm.at[cb.at[r]], rows[b], gsem[b]).wait()
                    pltpu.sync_copy(rows[b], acc.at[db.at[r]], add=True)
                    if r < CPB - 2:
                        pltpu.async_copy(y_hbm.at[cb.at[r + 2]], rows[b],
                                         gsem[b])
                    else:
                        rn = r + 2 - CPB

                        @pl.when(j + 1 < NBLK)
                        def _next_blk_gather():
                            if rn == 0:
                                nrow = row0 + (j + 1) * CPB
                                pltpu.make_async_copy(
                                    src_hbm.at[pl.ds(nrow, CPB)], nb,
                                    isem[1 - half]).wait()
                                pltpu.make_async_copy(
                                    dst_hbm.at[pl.ds(nrow, CPB)], ndb,
                                    isem[1 - half]).wait()
                            pltpu.async_copy(y_hbm.at[nb.at[rn]], rows[b],
                                             gsem[b])

                @pl.when(j + 2 < NBLK)
                def _load_next():
                    nrow = row0 + (j + 2) * CPB
                    pltpu.async_copy(src_hbm.at[pl.ds(nrow, CPB)], cb,
                                     isem[half])
                    pltpu.async_copy(dst_hbm.at[pl.ds(nrow, CPB)], db,
                                     isem[half])

            return carry

        lax.fori_loop(0, NBLK // 2, pair_step, 0)

        plsc.subcore_barrier()
        pltpu.sync_copy(acc.at[pl.ds(r0, RPT)], out_hbm.at[c, pl.ds(r0, RPT)])

    return functools.partial(
        pl.kernel,
        out_type=jax.ShapeDtypeStruct((NC, NPAD, D), jnp.float32),
        mesh=plsc.VectorSubcoreMesh(core_axis_name="c", subcore_axis_name="s"),
        compiler_params=pltpu.CompilerParams(use_tc_tiling_on_sc=False),
        scratch_types=[
            pltpu.VMEM_SHARED((NPAD, D), jnp.float32),
            pltpu.VMEM((CPB, K), jnp.int32),
            pltpu.VMEM((CPB, K), jnp.int32),
            pltpu.VMEM((CPB, K), jnp.int32),
            pltpu.VMEM((CPB, K), jnp.int32),
            pltpu.VMEM((K, D), jnp.float32),
            pltpu.VMEM((K, D), jnp.float32),
            pltpu.SemaphoreType.DMA,
            pltpu.SemaphoreType.DMA,
            pltpu.SemaphoreType.DMA,
            pltpu.SemaphoreType.DMA,
        ],
    )(body)


_prop128 = _make_propagate(D_IN)
_prop64 = _make_propagate(H2)


def _prescale_body(cnt_ref, x_ref, dinv_ref, y_ref):
    deg = 1.0 + cnt_ref[0, :, 0:1] + cnt_ref[1, :, 0:1]
    dv = lax.rsqrt(deg)
    dinv_ref[...] = dv
    y_ref[...] = x_ref[...] * dv


_prescale = pl.pallas_call(
    _prescale_body,
    grid=(NPAD // RB,),
    in_specs=[
        pl.BlockSpec((NC, RB, DEG_W), lambda i: (0, i, 0)),
        pl.BlockSpec((RB, D_IN), lambda i: (i, 0)),
    ],
    out_specs=[
        pl.BlockSpec((RB, 1), lambda i: (i, 0)),
        pl.BlockSpec((RB, D_IN), lambda i: (i, 0)),
    ],
    out_shape=[
        jax.ShapeDtypeStruct((NPAD, 1), jnp.float32),
        jax.ShapeDtypeStruct((NPAD, D_IN), jnp.float32),
    ],
)


def _dense1_body(a_ref, y_ref, dinv_ref, w1_ref, b1_ref, w2_ref, out_ref):
    dv = dinv_ref[...]
    p = dv * (a_ref[0] + a_ref[1] + y_ref[...])
    z = jnp.dot(p, w1_ref[...], preferred_element_type=jnp.float32) + b1_ref[...]
    h = jnp.maximum(z, 0.0)
    g = jnp.dot(h, w2_ref[...], preferred_element_type=jnp.float32)
    out_ref[...] = dv * g


_dense1 = pl.pallas_call(
    _dense1_body,
    grid=(NPAD // RB,),
    in_specs=[
        pl.BlockSpec((NC, RB, D_IN), lambda i: (0, i, 0)),
        pl.BlockSpec((RB, D_IN), lambda i: (i, 0)),
        pl.BlockSpec((RB, 1), lambda i: (i, 0)),
        pl.BlockSpec((D_IN, H1), lambda i: (0, 0)),
        pl.BlockSpec((1, H1), lambda i: (0, 0)),
        pl.BlockSpec((H1, H2), lambda i: (0, 0)),
    ],
    out_specs=pl.BlockSpec((RB, H2), lambda i: (i, 0)),
    out_shape=jax.ShapeDtypeStruct((NPAD, H2), jnp.float32),
)


def _dense2_body(a_ref, y_ref, dinv_ref, b2_ref, wfc_ref, bfc_ref, out_ref):
    dv = dinv_ref[...]
    z = dv * (a_ref[0] + a_ref[1] + y_ref[...]) + b2_ref[...]
    out_ref[...] = (
        jnp.dot(z, wfc_ref[...], preferred_element_type=jnp.float32) + bfc_ref[...]
    )


RB2 = 1000

_dense2 = pl.pallas_call(
    _dense2_body,
    grid=(N // RB2,),
    in_specs=[
        pl.BlockSpec((NC, RB2, H2), lambda i: (0, i, 0)),
        pl.BlockSpec((RB2, H2), lambda i: (i, 0)),
        pl.BlockSpec((RB2, 1), lambda i: (i, 0)),
        pl.BlockSpec((1, H2), lambda i: (0, 0)),
        pl.BlockSpec((H2, C_OUT), lambda i: (0, 0)),
        pl.BlockSpec((1, C_OUT), lambda i: (0, 0)),
    ],
    out_specs=pl.BlockSpec((RB2, C_OUT), lambda i: (i, 0)),
    out_shape=jax.ShapeDtypeStruct((N, C_OUT), jnp.float32),
)


def kernel(x, edge_index, W1, b1, W2, b2, Wfc, bfc):
    ei = edge_index.astype(jnp.int32)
    pad_idx = N + jnp.arange(E2 - E, dtype=jnp.int32) % (NPAD - N)
    ei = jnp.concatenate([ei, jnp.stack([pad_idx, pad_idx])], axis=1)
    src2d = ei[0].reshape(EROWS, K)
    dst2d = ei[1].reshape(EROWS, K)
    xp = jnp.pad(x, ((0, NPAD - N), (0, 0)))

    counts = _deg_kernel(dst2d)
    dinv, y1 = _prescale(counts, xp)

    prop1 = _prop128(y1, src2d, dst2d)
    y2 = _dense1(prop1, y1, dinv, W1, b1.reshape(1, H1), W2)

    prop2 = _prop64(y2, src2d, dst2d)
    out = _dense2(prop2, y2, dinv, b2.reshape(1, H2), Wfc, bfc.reshape(1, C_OUT))
    return out


# speedup vs baseline: 1.2434x; 1.0280x over previous
import functools

import jax
import jax.numpy as jnp
from jax import lax
from jax.experimental import pallas as pl
from jax.experimental.pallas import tpu as pltpu
from jax.experimental.pallas import tpu_sc as plsc

N = 10000
NPAD = 10240
E = 320000
D_IN = 128
H1 = 256
H2 = 64
C_OUT = 40

NC = 2
NS = 16
NW = NC * NS
EPT = E // NW
K = 128
NCH = EPT // K
TAIL = EPT - NCH * K
RPT = NPAD // NS

RB = 2048


DEG_W = 16
DEG_LAG = 8


def _deg_body(ei_hbm, out_hbm, acc, dbuf, ones_rows, zbuf, asem):
    c = lax.axis_index("c")
    s = lax.axis_index("s")
    w = c * NS + s
    r0 = s * RPT

    ones16 = jnp.ones((16,), jnp.float32)
    zeros16 = jnp.zeros((16,), jnp.float32)

    def fill_step(j, carry):
        ones_rows[j] = ones16
        return carry

    lax.fori_loop(0, K, fill_step, 0)

    def zfill_step(j, carry):
        zbuf[j] = zeros16
        return carry

    lax.fori_loop(0, 64, zfill_step, 0)

    def zcopy_step(j, carry):
        pltpu.sync_copy(zbuf, acc.at[pl.ds(r0 + j * 64, 64)])
        return carry

    lax.fori_loop(0, RPT // 64, zcopy_step, 0)

    pltpu.sync_copy(ei_hbm.at[1, pl.ds(w * EPT, EPT)], dbuf)
    plsc.subcore_barrier()

    def add_step(i, carry):
        pltpu.async_copy(ones_rows, acc.at[dbuf.at[pl.ds(i * K, K)]], asem,
                         add=True)

        @pl.when(i >= DEG_LAG)
        def _drain_one():
            pltpu.make_async_copy(ones_rows, acc.at[dbuf.at[pl.ds(0, K)]],
                                  asem).wait()

        return carry

    lax.fori_loop(0, NCH, add_step, 0)

    pltpu.async_copy(ones_rows.at[pl.ds(0, TAIL)],
                     acc.at[dbuf.at[pl.ds(NCH * K, TAIL)]], asem, add=True)
    pltpu.make_async_copy(ones_rows.at[pl.ds(0, TAIL)],
                          acc.at[dbuf.at[pl.ds(0, TAIL)]], asem).wait()

    def drain_step(i, carry):
        pltpu.make_async_copy(ones_rows, acc.at[dbuf.at[pl.ds(0, K)]],
                              asem).wait()
        return carry

    lax.fori_loop(0, DEG_LAG, drain_step, 0)

    plsc.subcore_barrier()
    pltpu.sync_copy(acc.at[pl.ds(r0, RPT)], out_hbm.at[c, pl.ds(r0, RPT)])


_deg_kernel = functools.partial(
    pl.kernel,
    out_type=jax.ShapeDtypeStruct((NC, NPAD, DEG_W), jnp.float32),
    mesh=plsc.VectorSubcoreMesh(core_axis_name="c", subcore_axis_name="s"),
    compiler_params=pltpu.CompilerParams(use_tc_tiling_on_sc=False),
    scratch_types=[
        pltpu.VMEM_SHARED((NPAD, DEG_W), jnp.float32),
        pltpu.VMEM((EPT,), jnp.int32),
        pltpu.VMEM((K, DEG_W), jnp.float32),
        pltpu.VMEM((64, DEG_W), jnp.float32),
        pltpu.SemaphoreType.DMA,
    ],
)(_deg_body)


def _make_propagate(D):
    def body(y_hbm, ei_hbm, out_hbm, acc,
             sidx, didx, tsidx, tdidx, rows0, rows1,
             isems, isemd, gsem0, gsem1):
        c = lax.axis_index("c")
        s = lax.axis_index("s")
        w = c * NS + s
        r0 = s * RPT
        e0 = w * EPT

        zeros16 = jnp.zeros((16,), jnp.float32)

        def zfill_step(q, carry):
            rows0[q // (D // 16), pl.ds((q % (D // 16)) * 16, 16)] = zeros16
            return carry

        lax.fori_loop(0, K * D // 16, zfill_step, 0)

        def zcopy_step(q, carry):
            pltpu.sync_copy(rows0, acc.at[pl.ds(r0 + q * K, K)])
            return carry

        lax.fori_loop(0, RPT // K, zcopy_step, 0)

        rows = (rows0, rows1)
        gsem = (gsem0, gsem1)

        def ld_s(i, q):
            pltpu.async_copy(ei_hbm.at[0, pl.ds(e0 + i * K, K)], sidx.at[q],
                             isems.at[q])

        def wt_s(i, q):
            pltpu.make_async_copy(ei_hbm.at[0, pl.ds(e0 + i * K, K)],
                                  sidx.at[q], isems.at[q]).wait()

        def ld_d(i, q):
            pltpu.async_copy(ei_hbm.at[1, pl.ds(e0 + i * K, K)], didx.at[q],
                             isemd.at[q])

        def wt_d(i, q):
            pltpu.make_async_copy(ei_hbm.at[1, pl.ds(e0 + i * K, K)],
                                  didx.at[q], isemd.at[q]).wait()

        for q in range(4):
            ld_s(q, q)
            ld_d(q, q)
        pltpu.sync_copy(ei_hbm.at[0, pl.ds(e0 + NCH * K, TAIL)], tsidx)
        pltpu.sync_copy(ei_hbm.at[1, pl.ds(e0 + NCH * K, TAIL)], tdidx)
        plsc.subcore_barrier()
        wt_s(0, 0)
        pltpu.async_copy(y_hbm.at[sidx.at[0]], rows0, gsem0)
        wt_s(1, 1)
        pltpu.async_copy(y_hbm.at[sidx.at[1]], rows1, gsem1)

        def slot(i, b, q):
            pltpu.make_async_copy(y_hbm.at[sidx.at[q]], rows[b],
                                  gsem[b]).wait()
            wt_d(i, q)
            pltpu.sync_copy(rows[b], acc.at[didx.at[q]], add=True)

            @pl.when(i + 4 < NCH)
            def _reload():
                ld_s(i + 4, q)
                ld_d(i + 4, q)

            @pl.when(i + 2 < NCH)
            def _next_gather():
                wt_s(i + 2, (q + 2) % 4)
                pltpu.async_copy(y_hbm.at[sidx.at[(q + 2) % 4]], rows[b],
                                 gsem[b])

        def quad_step(p, carry):
            for r in range(4):
                slot(4 * p + r, r % 2, r)
            return carry

        lax.fori_loop(0, NCH // 4, quad_step, 0)
        for i in range(NCH - NCH % 4, NCH):
            slot(i, i % 2, i % 4)

        pltpu.async_copy(y_hbm.at[tsidx], rows0.at[pl.ds(0, TAIL)],
                         gsem0).wait()
        pltpu.sync_copy(rows0.at[pl.ds(0, TAIL)], acc.at[tdidx], add=True)

        plsc.subcore_barrier()
        pltpu.sync_copy(acc.at[pl.ds(r0, RPT)], out_hbm.at[c, pl.ds(r0, RPT)])

    return functools.partial(
        pl.kernel,
        out_type=jax.ShapeDtypeStruct((NC, NPAD, D), jnp.float32),
        mesh=plsc.VectorSubcoreMesh(core_axis_name="c", subcore_axis_name="s"),
        compiler_params=pltpu.CompilerParams(use_tc_tiling_on_sc=False),
        scratch_types=[
            pltpu.VMEM_SHARED((NPAD, D), jnp.float32),
            pltpu.VMEM((4, K), jnp.int32),
            pltpu.VMEM((4, K), jnp.int32),
            pltpu.VMEM((TAIL,), jnp.int32),
            pltpu.VMEM((TAIL,), jnp.int32),
            pltpu.VMEM((K, D), jnp.float32),
            pltpu.VMEM((K, D), jnp.float32),
            pltpu.SemaphoreType.DMA((4,)),
            pltpu.SemaphoreType.DMA((4,)),
            pltpu.SemaphoreType.DMA,
            pltpu.SemaphoreType.DMA,
        ],
    )(body)


_prop128 = _make_propagate(D_IN)
_prop64 = _make_propagate(H2)


def _prescale_body(cnt_ref, x_ref, dinv_ref, y_ref):
    deg = 1.0 + cnt_ref[0, :, 0:1] + cnt_ref[1, :, 0:1]
    dv = lax.rsqrt(deg)
    dinv_ref[...] = dv
    y_ref[...] = x_ref[...] * dv


_prescale = pl.pallas_call(
    _prescale_body,
    grid=(NPAD // RB,),
    in_specs=[
        pl.BlockSpec((NC, RB, DEG_W), lambda i: (0, i, 0)),
        pl.BlockSpec((RB, D_IN), lambda i: (i, 0)),
    ],
    out_specs=[
        pl.BlockSpec((RB, 1), lambda i: (i, 0)),
        pl.BlockSpec((RB, D_IN), lambda i: (i, 0)),
    ],
    out_shape=[
        jax.ShapeDtypeStruct((NPAD, 1), jnp.float32),
        jax.ShapeDtypeStruct((NPAD, D_IN), jnp.float32),
    ],
)


def _dense1_body(a_ref, y_ref, dinv_ref, w1_ref, b1_ref, w2_ref, out_ref):
    dv = dinv_ref[...]
    p = dv * (a_ref[0] + a_ref[1] + y_ref[...])
    z = jnp.dot(p, w1_ref[...], preferred_element_type=jnp.float32) + b1_ref[...]
    h = jnp.maximum(z, 0.0)
    g = jnp.dot(h, w2_ref[...], preferred_element_type=jnp.float32)
    out_ref[...] = dv * g


_dense1 = pl.pallas_call(
    _dense1_body,
    grid=(NPAD // RB,),
    in_specs=[
        pl.BlockSpec((NC, RB, D_IN), lambda i: (0, i, 0)),
        pl.BlockSpec((RB, D_IN), lambda i: (i, 0)),
        pl.BlockSpec((RB, 1), lambda i: (i, 0)),
        pl.BlockSpec((D_IN, H1), lambda i: (0, 0)),
        pl.BlockSpec((1, H1), lambda i: (0, 0)),
        pl.BlockSpec((H1, H2), lambda i: (0, 0)),
    ],
    out_specs=pl.BlockSpec((RB, H2), lambda i: (i, 0)),
    out_shape=jax.ShapeDtypeStruct((NPAD, H2), jnp.float32),
)


def _dense2_body(a_ref, y_ref, dinv_ref, b2_ref, wfc_ref, bfc_ref, out_ref):
    dv = dinv_ref[...]
    z = dv * (a_ref[0] + a_ref[1] + y_ref[...]) + b2_ref[...]
    out_ref[...] = (
        jnp.dot(z, wfc_ref[...], preferred_element_type=jnp.float32) + bfc_ref[...]
    )


RB2 = 1000

_dense2 = pl.pallas_call(
    _dense2_body,
    grid=(N // RB2,),
    in_specs=[
        pl.BlockSpec((NC, RB2, H2), lambda i: (0, i, 0)),
        pl.BlockSpec((RB2, H2), lambda i: (i, 0)),
        pl.BlockSpec((RB2, 1), lambda i: (i, 0)),
        pl.BlockSpec((1, H2), lambda i: (0, 0)),
        pl.BlockSpec((H2, C_OUT), lambda i: (0, 0)),
        pl.BlockSpec((1, C_OUT), lambda i: (0, 0)),
    ],
    out_specs=pl.BlockSpec((RB2, C_OUT), lambda i: (i, 0)),
    out_shape=jax.ShapeDtypeStruct((N, C_OUT), jnp.float32),
)


def kernel(x, edge_index, W1, b1, W2, b2, Wfc, bfc):
    ei = edge_index.astype(jnp.int32)
    xp = jnp.pad(x, ((0, NPAD - N), (0, 0)))

    counts = _deg_kernel(ei)
    dinv, y1 = _prescale(counts, xp)

    prop1 = _prop128(y1, ei)
    y2 = _dense1(prop1, y1, dinv, W1, b1.reshape(1, H1), W2)

    prop2 = _prop64(y2, ei)
    out = _dense2(prop2, y2, dinv, b2.reshape(1, H2), Wfc, bfc.reshape(1, C_OUT))
    return out
